# baseline (speedup 1.0000x reference)
import functools

import jax
import jax.numpy as jnp
from jax import lax
from jax.experimental import pallas as pl
from jax.experimental.pallas import tpu as pltpu
from jax.experimental.pallas import tpu_sc as plsc

_LANE = 128
_GRP = 16


def kernel(user_id, table):
    (batch,) = user_id.shape
    vocab, dim = table.shape

    info = plsc.get_sparse_core_info()
    n_workers = info.num_cores * info.num_subcores
    b_per_w = batch // n_workers

    idx = user_id.astype(jnp.int32)
    table_t = table.T

    @functools.partial(
        pl.kernel,
        out_type=jax.ShapeDtypeStruct((dim, batch), table.dtype),
        mesh=plsc.VectorSubcoreMesh(core_axis_name="c", subcore_axis_name="s"),
        scratch_types=[
            pltpu.VMEM((b_per_w,), jnp.int32),
            pltpu.VMEM((_GRP, dim, _LANE), table.dtype),
            pltpu.VMEM((_GRP, dim, _LANE), table.dtype),
            pltpu.VMEM((_GRP, dim, _LANE), table.dtype),
            pltpu.VMEM((dim, b_per_w), table.dtype),
            pltpu.SemaphoreType.DMA,
            pltpu.SemaphoreType.DMA,
            pltpu.SemaphoreType.DMA,
        ],
        compiler_params=pltpu.CompilerParams(
            use_tc_tiling_on_sc=True, needs_layout_passes=False
        ),
    )
    def _gather(idx_hbm, tt_hbm, outt_hbm, idx_v, win_a, win_b, win_c, res_v,
                sem_a, sem_b, sem_c):
        wid = lax.axis_index("s") * info.num_cores + lax.axis_index("c")
        base = wid * b_per_w
        pltpu.sync_copy(idx_hbm.at[pl.ds(base, b_per_w)], idx_v)
        lanes = lax.iota(jnp.int32, 16)

        def fire(g, win, sem):
            vec = idx_v[pl.ds(g * _GRP, _GRP)]
            for j in range(_GRP):
                blk = pl.multiple_of((vec[j] // _LANE) * _LANE, _LANE)
                pltpu.async_copy(
                    tt_hbm.at[:, pl.ds(blk, _LANE)], win.at[j], sem
                )

        def drain_extract(g, win, sem):
            for j in range(_GRP):
                pltpu.make_async_copy(
                    tt_hbm.at[:, pl.ds(0, _LANE)], win.at[j], sem
                ).wait()
            vec = idx_v[pl.ds(g * _GRP, _GRP)]
            cols = vec - (vec // _LANE) * _LANE
            slots = g * _GRP + lanes
            for c in range(dim):
                crow = jnp.full((16,), c, jnp.int32)
                vals = plsc.load_gather(win, [lanes, crow, cols])
                plsc.store_scatter(res_v, [crow, slots], vals)

        fire(0, win_a, sem_a)
        fire(1, win_b, sem_b)

        def body(p, carry):
            g = 3 * p
            fire(g + 2, win_c, sem_c)
            drain_extract(g, win_a, sem_a)
            fire(g + 3, win_a, sem_a)
            drain_extract(g + 1, win_b, sem_b)
            fire(g + 4, win_b, sem_b)
            drain_extract(g + 2, win_c, sem_c)
            return carry

        n_groups = b_per_w // _GRP
        lax.fori_loop(0, (n_groups - 2) // 3, body, 0)
        drain_extract(n_groups - 2, win_a, sem_a)
        drain_extract(n_groups - 1, win_b, sem_b)
        pltpu.sync_copy(res_v, outt_hbm.at[:, pl.ds(base, b_per_w)])

    return _gather(idx, table_t).T


# speedup vs baseline: 1.0258x; 1.0258x over previous
import functools

import jax
import jax.numpy as jnp
from jax import lax
from jax.experimental import pallas as pl
from jax.experimental.pallas import tpu as pltpu
from jax.experimental.pallas import tpu_sc as plsc

_LANE = 128
_GRP = 16


def kernel(user_id, table):
    (batch,) = user_id.shape
    vocab, dim = table.shape

    info = plsc.get_sparse_core_info()
    n_workers = info.num_cores * info.num_subcores
    b_per_w = batch // n_workers

    idx = user_id.astype(jnp.int32)
    table_t = table.T

    @functools.partial(
        pl.kernel,
        out_type=jax.ShapeDtypeStruct((dim, batch), table.dtype),
        mesh=plsc.VectorSubcoreMesh(core_axis_name="c", subcore_axis_name="s"),
        scratch_types=[
            pltpu.VMEM((b_per_w,), jnp.int32),
            pltpu.VMEM((dim, _GRP * _LANE), table.dtype),
            pltpu.VMEM((dim, _GRP * _LANE), table.dtype),
            pltpu.VMEM((dim, _GRP * _LANE), table.dtype),
            pltpu.VMEM((dim, b_per_w), table.dtype),
            pltpu.SemaphoreType.DMA,
            pltpu.SemaphoreType.DMA,
            pltpu.SemaphoreType.DMA,
        ],
        compiler_params=pltpu.CompilerParams(
            use_tc_tiling_on_sc=True, needs_layout_passes=False
        ),
    )
    def _gather(idx_hbm, tt_hbm, outt_hbm, idx_v, win_a, win_b, win_c, res_v,
                sem_a, sem_b, sem_c):
        wid = lax.axis_index("s") * info.num_cores + lax.axis_index("c")
        base = wid * b_per_w
        pltpu.sync_copy(idx_hbm.at[pl.ds(base, b_per_w)], idx_v)
        lanes = lax.iota(jnp.int32, 16)

        def fire(g, win, sem):
            vec = idx_v[pl.ds(g * _GRP, _GRP)]
            for j in range(_GRP):
                blk = pl.multiple_of((vec[j] // _LANE) * _LANE, _LANE)
                pltpu.async_copy(
                    tt_hbm.at[:, pl.ds(blk, _LANE)],
                    win.at[:, pl.ds(j * _LANE, _LANE)],
                    sem,
                )

        def drain_extract(g, win, sem):
            pltpu.make_async_copy(
                tt_hbm.at[:, pl.ds(0, _GRP * _LANE)], win, sem
            ).wait()
            vec = idx_v[pl.ds(g * _GRP, _GRP)]
            cols = lanes * _LANE + (vec - (vec // _LANE) * _LANE)
            slots = g * _GRP + lanes
            for c in range(dim):
                crow = jnp.full((16,), c, jnp.int32)
                vals = plsc.load_gather(win, [crow, cols])
                plsc.store_scatter(res_v, [crow, slots], vals)

        fire(0, win_a, sem_a)
        fire(1, win_b, sem_b)

        def body(p, carry):
            g = 3 * p
            fire(g + 2, win_c, sem_c)
            drain_extract(g, win_a, sem_a)
            fire(g + 3, win_a, sem_a)
            drain_extract(g + 1, win_b, sem_b)
            fire(g + 4, win_b, sem_b)
            drain_extract(g + 2, win_c, sem_c)
            return carry

        n_groups = b_per_w // _GRP
        lax.fori_loop(0, (n_groups - 2) // 3, body, 0)
        drain_extract(n_groups - 2, win_a, sem_a)
        drain_extract(n_groups - 1, win_b, sem_b)
        pltpu.sync_copy(res_v, outt_hbm.at[:, pl.ds(base, b_per_w)])

    return _gather(idx, table_t).T
